# baseline (speedup 1.0000x reference)
import jax
import jax.numpy as jnp
from jax import lax
from jax.experimental import pallas as pl
from jax.experimental.pallas import tpu as pltpu, tpu_sc as plsc

EMBED = 64
BATCH = 16384
NUM_CORES = 2
NUM_SUBCORES = 16
LANES = 16
NUM_WORKERS = NUM_CORES * NUM_SUBCORES
ROWS_PER_WORKER = BATCH // NUM_WORKERS
CHUNK = 128
NUM_CHUNKS = ROWS_PER_WORKER // CHUNK
VECS_PER_ROW = EMBED // LANES


def _body(users_hbm, items_hbm, utab_hbm, itab_hbm, out_hbm,
          idx_u, idx_i, rows_u, rows_i, out_v, sems_u, sems_i):
    wid = lax.axis_index("s") * NUM_CORES + lax.axis_index("c")
    base = wid * ROWS_PER_WORKER

    for c in range(NUM_CHUNKS):
        pltpu.sync_copy(users_hbm.at[pl.ds(base + c * CHUNK, CHUNK)],
                        idx_u.at[c])
        pltpu.sync_copy(items_hbm.at[pl.ds(base + c * CHUNK, CHUNK)],
                        idx_i.at[c])

    def start(c):
        slot = c % 2
        du = pltpu.async_copy(utab_hbm.at[idx_u.at[c]], rows_u.at[slot],
                              sems_u[slot])
        di = pltpu.async_copy(itab_hbm.at[idx_i.at[c]], rows_i.at[slot],
                              sems_i[slot])
        return du, di

    lane = lax.iota(jnp.int32, LANES)
    perms = [lane ^ sh for sh in (8, 4, 2, 1)]

    def dyn_perm(a, perm):
        return lax.gather(
            a, perm[:, None],
            lax.GatherDimensionNumbers(offset_dims=(),
                                       collapsed_slice_dims=(0,),
                                       start_index_map=(0,)),
            slice_sizes=(1,), mode=lax.GatherScatterMode.PROMISE_IN_BOUNDS)

    def compute(c):
        slot = c % 2
        ru = rows_u.at[slot]
        ri = rows_i.at[slot]

        def group(g, carry):
            res = jnp.zeros((LANES,), jnp.float32)
            for j in range(LANES):
                r = g * LANES + j
                acc = (ru[r, pl.ds(0, LANES)] * ri[r, pl.ds(0, LANES)]
                       + ru[r, pl.ds(LANES, LANES)]
                       * ri[r, pl.ds(LANES, LANES)])
                acc = acc + (ru[r, pl.ds(2 * LANES, LANES)]
                             * ri[r, pl.ds(2 * LANES, LANES)]
                             + ru[r, pl.ds(3 * LANES, LANES)]
                             * ri[r, pl.ds(3 * LANES, LANES)])
                for p in perms:
                    acc = acc + dyn_perm(acc, p)
                res = jnp.where(lane == j, acc, res)
            out_v[pl.ds(c * CHUNK + g * LANES, LANES)] = res
            return carry

        lax.fori_loop(0, CHUNK // LANES, group, 0)

    descs = start(0)
    for c in range(NUM_CHUNKS):
        nxt = start(c + 1) if c + 1 < NUM_CHUNKS else None
        descs[0].wait()
        descs[1].wait()
        compute(c)
        descs = nxt

    pltpu.sync_copy(out_v, out_hbm.at[pl.ds(base, ROWS_PER_WORKER)])


@jax.jit
def _scores(users, items, user_table, item_table):
    mesh = plsc.VectorSubcoreMesh(core_axis_name="c", subcore_axis_name="s",
                                  num_cores=NUM_CORES,
                                  num_subcores=NUM_SUBCORES)
    return pl.kernel(
        _body,
        out_type=jax.ShapeDtypeStruct((BATCH,), jnp.float32),
        mesh=mesh,
        compiler_params=pltpu.CompilerParams(use_tc_tiling_on_sc=False),
        scratch_types=[
            pltpu.VMEM((NUM_CHUNKS, CHUNK), jnp.int32),
            pltpu.VMEM((NUM_CHUNKS, CHUNK), jnp.int32),
            pltpu.VMEM((2, CHUNK, EMBED), jnp.float32),
            pltpu.VMEM((2, CHUNK, EMBED), jnp.float32),
            pltpu.VMEM((ROWS_PER_WORKER,), jnp.float32),
            [pltpu.SemaphoreType.DMA, pltpu.SemaphoreType.DMA],
            [pltpu.SemaphoreType.DMA, pltpu.SemaphoreType.DMA],
        ],
    )(users, items, user_table, item_table)


def kernel(users, items, user_table, item_table):
    return _scores(users, items, user_table, item_table)


# speedup vs baseline: 1.5873x; 1.5873x over previous
import jax
import jax.numpy as jnp
from jax import lax
from jax.experimental import pallas as pl
from jax.experimental.pallas import tpu as pltpu, tpu_sc as plsc

EMBED = 64
BATCH = 16384
NUM_CORES = 2
NUM_SUBCORES = 16
LANES = 16
NUM_WORKERS = NUM_CORES * NUM_SUBCORES
ROWS_PER_WORKER = BATCH // NUM_WORKERS
CHUNK = 128
NUM_CHUNKS = ROWS_PER_WORKER // CHUNK


def _body(users_hbm, items_hbm, utab_hbm, itab_hbm, out_hbm,
          idx_u, idx_i, rows_u, rows_i, out_v, sems_u, sems_i):
    wid = lax.axis_index("s") * NUM_CORES + lax.axis_index("c")
    base = wid * ROWS_PER_WORKER

    pltpu.sync_copy(users_hbm.at[pl.ds(base, ROWS_PER_WORKER)], idx_u)
    pltpu.sync_copy(items_hbm.at[pl.ds(base, ROWS_PER_WORKER)], idx_i)

    lane = lax.iota(jnp.int32, LANES)
    perms = [lane ^ sh for sh in (8, 4, 2, 1)]

    def dyn_perm(a, perm):
        return lax.gather(
            a, perm[:, None],
            lax.GatherDimensionNumbers(offset_dims=(),
                                       collapsed_slice_dims=(0,),
                                       start_index_map=(0,)),
            slice_sizes=(1,), mode=lax.GatherScatterMode.PROMISE_IN_BOUNDS)

    def fire(c):
        slot = c % 2

        def grp(g, carry):
            uvec = idx_u[pl.ds(c * CHUNK + g * LANES, LANES)]
            ivec = idx_i[pl.ds(c * CHUNK + g * LANES, LANES)]
            for j in range(LANES):
                r = g * LANES + j
                pltpu.async_copy(utab_hbm.at[uvec[j]],
                                 rows_u.at[slot].at[r], sems_u[slot])
                pltpu.async_copy(itab_hbm.at[ivec[j]],
                                 rows_i.at[slot].at[r], sems_i[slot])
            return carry

        lax.fori_loop(0, CHUNK // LANES, grp, 0)

    def drain(c):
        slot = c % 2
        pltpu.make_async_copy(utab_hbm.at[pl.ds(0, CHUNK)],
                              rows_u.at[slot], sems_u[slot]).wait()
        pltpu.make_async_copy(itab_hbm.at[pl.ds(0, CHUNK)],
                              rows_i.at[slot], sems_i[slot]).wait()

    def compute(c):
        slot = c % 2
        ru = rows_u.at[slot]
        ri = rows_i.at[slot]

        def group(g, carry):
            res = jnp.zeros((LANES,), jnp.float32)
            for j in range(LANES):
                r = g * LANES + j
                acc = (ru[r, pl.ds(0, LANES)] * ri[r, pl.ds(0, LANES)]
                       + ru[r, pl.ds(LANES, LANES)]
                       * ri[r, pl.ds(LANES, LANES)])
                acc = acc + (ru[r, pl.ds(2 * LANES, LANES)]
                             * ri[r, pl.ds(2 * LANES, LANES)]
                             + ru[r, pl.ds(3 * LANES, LANES)]
                             * ri[r, pl.ds(3 * LANES, LANES)])
                for p in perms:
                    acc = acc + dyn_perm(acc, p)
                res = jnp.where(lane == j, acc, res)
            out_v[pl.ds(c * CHUNK + g * LANES, LANES)] = res
            return carry

        lax.fori_loop(0, CHUNK // LANES, group, 0)

    fire(0)
    for c in range(NUM_CHUNKS):
        if c + 1 < NUM_CHUNKS:
            fire(c + 1)
        drain(c)
        compute(c)

    pltpu.sync_copy(out_v, out_hbm.at[pl.ds(base, ROWS_PER_WORKER)])


@jax.jit
def _scores(users, items, user_table, item_table):
    mesh = plsc.VectorSubcoreMesh(core_axis_name="c", subcore_axis_name="s",
                                  num_cores=NUM_CORES,
                                  num_subcores=NUM_SUBCORES)
    return pl.kernel(
        _body,
        out_type=jax.ShapeDtypeStruct((BATCH,), jnp.float32),
        mesh=mesh,
        scratch_types=[
            pltpu.VMEM((ROWS_PER_WORKER,), jnp.int32),
            pltpu.VMEM((ROWS_PER_WORKER,), jnp.int32),
            pltpu.VMEM((2, CHUNK, EMBED), jnp.float32),
            pltpu.VMEM((2, CHUNK, EMBED), jnp.float32),
            pltpu.VMEM((ROWS_PER_WORKER,), jnp.float32),
            [pltpu.SemaphoreType.DMA, pltpu.SemaphoreType.DMA],
            [pltpu.SemaphoreType.DMA, pltpu.SemaphoreType.DMA],
        ],
    )(users, items, user_table, item_table)


def kernel(users, items, user_table, item_table):
    return _scores(users, items, user_table, item_table)


# speedup vs baseline: 3.0143x; 1.8990x over previous
import jax
import jax.numpy as jnp
from jax import lax
from jax.experimental import pallas as pl
from jax.experimental.pallas import tpu as pltpu, tpu_sc as plsc

EMBED = 64
BATCH = 16384
NUM_ROWS = 1000000
NUM_CORES = 2
NUM_SUBCORES = 16
LANES = 16

HALF = 499968
WIN = 11904
NUM_WIN = HALF // WIN
TAIL_BASE = 2 * HALF
TAIL_ROWS = NUM_ROWS - TAIL_BASE
POS_PER_SUB = BATCH // NUM_SUBCORES
TAIL_BUCKET = NUM_WIN
DUMP_BUCKET = NUM_WIN + 1
NUM_BUCKETS = NUM_WIN + 2
CHUNK_E = 64

ROWS_PER_W2 = BATCH // (NUM_CORES * NUM_SUBCORES)
CHUNK2 = 128


def _gather_body(users_hbm, items_hbm, tabt_u, tabt_i, tail_u, tail_i,
                 stage_u, stage_i,
                 idx_v, wv_v, ev_v, tailv, tmp, ib, win_a, win_b, sm,
                 sem_a, sem_b, sem_g, sem_w):
    c = lax.axis_index("c")
    s = lax.axis_index("s")
    lane = lax.iota(jnp.int32, LANES)
    gbase = [(lane + k * LANES) * WIN for k in range(4)]

    for idx_hbm, tabt, tail, stage in (
            (users_hbm, tabt_u, tail_u, stage_u),
            (items_hbm, tabt_i, tail_i, stage_i)):
        pltpu.sync_copy(tail, tailv)
        pltpu.sync_copy(idx_hbm.at[pl.ds(s * POS_PER_SUB, POS_PER_SUB)],
                        idx_v)

        def vec_pass(j, carry):
            r = idx_v[pl.ds(j * LANES, LANES)]
            rloc = r - c * HALF
            limit = HALF + c * TAIL_ROWS
            keep = (rloc >= 0) & (rloc < limit)
            west = lax.div(rloc, jnp.int32(WIN))
            rl = rloc - west * WIN
            w_eff = jnp.where(keep, west, DUMP_BUCKET)
            pos = j * LANES + lane
            entry = pos * 16384 + rl
            wv_v[pl.ds(j * LANES, LANES)] = w_eff
            ev_v[pl.ds(j * LANES, LANES)] = entry
            return carry

        lax.fori_loop(0, POS_PER_SUB // LANES, vec_pass, 0)

        def zero_b(b, carry):
            sm[b] = 0
            return carry
        lax.fori_loop(0, NUM_BUCKETS, zero_b, 0)

        def count_pos(j, carry):
            wvec = wv_v[pl.ds(j * LANES, LANES)]
            for jj in range(LANES):
                w = wvec[jj]
                sm[w] = sm[w] + 1
            return carry
        lax.fori_loop(0, POS_PER_SUB // LANES, count_pos, 0)

        ST = NUM_BUCKETS
        CU = NUM_BUCKETS + NUM_BUCKETS + 1
        sm[ST] = 0
        def pfx(b, carry):
            sm[ST + b + 1] = sm[ST + b] + sm[b]
            sm[CU + b] = sm[ST + b]
            return carry
        lax.fori_loop(0, NUM_BUCKETS, pfx, 0)

        LB = CU + NUM_BUCKETS
        def place_pos(j, carry):
            wvec = wv_v[pl.ds(j * LANES, LANES)]
            evec = ev_v[pl.ds(j * LANES, LANES)]
            for jj in range(LANES):
                w = wvec[jj]
                a = sm[CU + w]
                sm[LB + a] = evec[jj]
                sm[CU + w] = a + 1
            return carry
        lax.fori_loop(0, POS_PER_SUB // LANES, place_pos, 0)

        base_r = pl.multiple_of(c * HALF, 128)

        def do_fire(w, winbuf, sem):
            off = pl.multiple_of(base_r + w * WIN, 128)
            for u in range(4):
                d = s * 4 + u
                pltpu.async_copy(tabt.at[d, pl.ds(off, WIN)],
                                 winbuf.at[pl.ds(d * WIN, WIN)],
                                 sem)

        def wait_own(sem):
            for u in range(4):
                pltpu.make_async_copy(
                    tabt.at[0, pl.ds(0, WIN)],
                    win_a.at[pl.ds(0, WIN)], sem).wait()

        def extract(w, winbuf):
            n0 = sm[ST + w]
            n1 = sm[ST + w + 1]
            nch = (n1 - n0 + (CHUNK_E - 1)) // CHUNK_E

            def chunk(ck, carry):
                e0 = n0 + ck * CHUNK_E
                nthis = jnp.minimum(CHUNK_E, n1 - e0)

                def gfire(t, carry2):
                    pk = sm[LB + e0 + t]
                    p_loc = pk // 16384
                    rl = pk - p_loc * 16384
                    for k in range(4):
                        ib[pl.ds(t * EMBED + k * LANES, LANES)] = \
                            gbase[k] + rl
                    pltpu.async_copy(
                        winbuf.at[ib.at[pl.ds(t * EMBED, EMBED)]],
                        tmp.at[t], sem_g)
                    return carry2
                lax.fori_loop(0, nthis, gfire, 0)

                def gdrain(t, carry2):
                    pltpu.make_async_copy(
                        winbuf.at[ib.at[pl.ds(0, EMBED)]],
                        tmp.at[0], sem_g).wait()
                    return carry2
                lax.fori_loop(0, nthis, gdrain, 0)

                def wfire(t, carry2):
                    pk = sm[LB + e0 + t]
                    p_loc = pk // 16384
                    p_glob = s * POS_PER_SUB + p_loc
                    pltpu.async_copy(tmp.at[t], stage.at[p_glob], sem_w)
                    return carry2
                lax.fori_loop(0, nthis, wfire, 0)

                def wdrain(t, carry2):
                    pltpu.make_async_copy(
                        tmp.at[0], stage.at[0], sem_w).wait()
                    return carry2
                lax.fori_loop(0, nthis, wdrain, 0)
                return carry
            lax.fori_loop(0, nch, chunk, 0)

        do_fire(0, win_a, sem_a)
        do_fire(1, win_b, sem_b)

        def pair(q, carry):
            w0 = 2 * q
            wait_own(sem_a)
            plsc.subcore_barrier()
            extract(w0, win_a)
            plsc.subcore_barrier()
            do_fire(w0 + 2, win_a, sem_a)
            wait_own(sem_b)
            plsc.subcore_barrier()
            extract(w0 + 1, win_b)
            plsc.subcore_barrier()
            do_fire(w0 + 3, win_b, sem_b)
            return carry

        lax.fori_loop(0, NUM_WIN // 2 - 1, pair, 0)
        wait_own(sem_a)
        plsc.subcore_barrier()
        extract(NUM_WIN - 2, win_a)
        wait_own(sem_b)
        plsc.subcore_barrier()
        extract(NUM_WIN - 1, win_b)

        n0 = sm[ST + TAIL_BUCKET]
        n1 = sm[ST + TAIL_BUCKET + 1]

        def tail_ent(t, carry):
            pk = sm[LB + n0 + t]
            p_loc = pk // 16384
            rl = pk - p_loc * 16384
            p_glob = s * POS_PER_SUB + p_loc
            pltpu.async_copy(tailv.at[rl], stage.at[p_glob], sem_w)
            pltpu.make_async_copy(tailv.at[0], stage.at[0],
                                  sem_w).wait()
            return carry
        lax.fori_loop(0, n1 - n0, tail_ent, 0)

        plsc.subcore_barrier()


def _dot_body(stage_u, stage_i, out_hbm, rows_u, rows_i, out_v,
              sems_u, sems_i):
    wid = lax.axis_index("s") * NUM_CORES + lax.axis_index("c")
    base = wid * ROWS_PER_W2
    lane = lax.iota(jnp.int32, LANES)
    perms = [lane ^ sh for sh in (8, 4, 2, 1)]

    def dyn_perm(a, perm):
        return lax.gather(
            a, perm[:, None],
            lax.GatherDimensionNumbers(offset_dims=(),
                                       collapsed_slice_dims=(0,),
                                       start_index_map=(0,)),
            slice_sizes=(1,), mode=lax.GatherScatterMode.PROMISE_IN_BOUNDS)

    def start(cn):
        slot = cn % 2
        du = pltpu.async_copy(stage_u.at[pl.ds(base + cn * CHUNK2, CHUNK2)],
                              rows_u.at[slot], sems_u[slot])
        di = pltpu.async_copy(stage_i.at[pl.ds(base + cn * CHUNK2, CHUNK2)],
                              rows_i.at[slot], sems_i[slot])
        return du, di

    def compute(cn):
        slot = cn % 2
        ru = rows_u.at[slot]
        ri = rows_i.at[slot]

        def group(g, carry):
            res = jnp.zeros((LANES,), jnp.float32)
            for j in range(LANES):
                r = g * LANES + j
                acc = (ru[r, pl.ds(0, LANES)] * ri[r, pl.ds(0, LANES)]
                       + ru[r, pl.ds(LANES, LANES)]
                       * ri[r, pl.ds(LANES, LANES)])
                acc = acc + (ru[r, pl.ds(2 * LANES, LANES)]
                             * ri[r, pl.ds(2 * LANES, LANES)]
                             + ru[r, pl.ds(3 * LANES, LANES)]
                             * ri[r, pl.ds(3 * LANES, LANES)])
                for p in perms:
                    acc = acc + dyn_perm(acc, p)
                res = jnp.where(lane == j, acc, res)
            out_v[pl.ds(cn * CHUNK2 + g * LANES, LANES)] = res
            return carry

        lax.fori_loop(0, CHUNK2 // LANES, group, 0)

    descs = start(0)
    for cn in range(ROWS_PER_W2 // CHUNK2):
        nxt = start(cn + 1) if cn + 1 < ROWS_PER_W2 // CHUNK2 else None
        descs[0].wait()
        descs[1].wait()
        compute(cn)
        descs = nxt

    pltpu.sync_copy(out_v, out_hbm.at[pl.ds(base, ROWS_PER_W2)])


@jax.jit
def _scores(users, items, user_table, item_table):
    mesh = plsc.VectorSubcoreMesh(core_axis_name="c", subcore_axis_name="s",
                                  num_cores=NUM_CORES,
                                  num_subcores=NUM_SUBCORES)
    tabt_u = user_table.T
    tabt_i = item_table.T
    tail_u = lax.slice(user_table, (TAIL_BASE, 0), (NUM_ROWS, EMBED))
    tail_i = lax.slice(item_table, (TAIL_BASE, 0), (NUM_ROWS, EMBED))

    stage_u, stage_i = pl.kernel(
        _gather_body,
        out_type=(jax.ShapeDtypeStruct((BATCH, EMBED), jnp.float32),
                  jax.ShapeDtypeStruct((BATCH, EMBED), jnp.float32)),
        mesh=mesh,
        scratch_types=[
            pltpu.VMEM((POS_PER_SUB,), jnp.int32),
            pltpu.VMEM((POS_PER_SUB,), jnp.int32),
            pltpu.VMEM((POS_PER_SUB,), jnp.int32),
            pltpu.VMEM((TAIL_ROWS, EMBED), jnp.float32),
            pltpu.VMEM((CHUNK_E, EMBED), jnp.float32),
            pltpu.VMEM((CHUNK_E * EMBED,), jnp.int32),
            pltpu.VMEM_SHARED((EMBED * WIN,), jnp.float32),
            pltpu.VMEM_SHARED((EMBED * WIN,), jnp.float32),
            pltpu.SMEM((NUM_BUCKETS * 2 + 2 + POS_PER_SUB,), jnp.int32),
            pltpu.SemaphoreType.DMA,
            pltpu.SemaphoreType.DMA,
            pltpu.SemaphoreType.DMA,
            pltpu.SemaphoreType.DMA,
        ],
    )(users, items, tabt_u, tabt_i, tail_u, tail_i)

    return pl.kernel(
        _dot_body,
        out_type=jax.ShapeDtypeStruct((BATCH,), jnp.float32),
        mesh=mesh,
        scratch_types=[
            pltpu.VMEM((2, CHUNK2, EMBED), jnp.float32),
            pltpu.VMEM((2, CHUNK2, EMBED), jnp.float32),
            pltpu.VMEM((ROWS_PER_W2,), jnp.float32),
            [pltpu.SemaphoreType.DMA, pltpu.SemaphoreType.DMA],
            [pltpu.SemaphoreType.DMA, pltpu.SemaphoreType.DMA],
        ],
    )(stage_u, stage_i)


def kernel(users, items, user_table, item_table):
    return _scores(users, items, user_table, item_table)


# speedup vs baseline: 3.1265x; 1.0372x over previous
import jax
import jax.numpy as jnp
from jax import lax
from jax.experimental import pallas as pl
from jax.experimental.pallas import tpu as pltpu, tpu_sc as plsc

EMBED = 64
BATCH = 16384
NUM_ROWS = 1000000
NUM_CORES = 2
NUM_SUBCORES = 16
LANES = 16

HALF = 499968
WIN = 11904
NUM_WIN = HALF // WIN
TAIL_BASE = 2 * HALF
TAIL_ROWS = NUM_ROWS - TAIL_BASE
POS_PER_SUB = BATCH // NUM_SUBCORES
TAIL_BUCKET = NUM_WIN
DUMP_BUCKET = NUM_WIN + 1
NUM_BUCKETS = NUM_WIN + 2
CHUNK_E = 64

ROWS_PER_W2 = BATCH // (NUM_CORES * NUM_SUBCORES)
CHUNK2 = 128


def _gather_body(users_hbm, items_hbm, tabt_u, tabt_i, tail_u, tail_i,
                 stage_u, stage_i,
                 idx_v, wv_v, ev_v, tailv, tmp, ib, win_a, win_b, sm,
                 sem_a, sem_b, sem_g, sem_w):
    c = lax.axis_index("c")
    s = lax.axis_index("s")
    lane = lax.iota(jnp.int32, LANES)
    gbase = [(lane + k * LANES) * WIN for k in range(4)]

    for idx_hbm, tabt, tail, stage in (
            (users_hbm, tabt_u, tail_u, stage_u),
            (items_hbm, tabt_i, tail_i, stage_i)):
        base_r0 = pl.multiple_of(c * HALF, 128)
        for wpre, wbuf, wsem in ((0, win_a, sem_a), (1, win_b, sem_b)):
            off0 = pl.multiple_of(base_r0 + wpre * WIN, 128)
            for u in range(4):
                d = s * 4 + u
                pltpu.async_copy(tabt.at[d, pl.ds(off0, WIN)],
                                 wbuf.at[pl.ds(d * WIN, WIN)], wsem)

        pltpu.sync_copy(tail, tailv)
        pltpu.sync_copy(idx_hbm.at[pl.ds(s * POS_PER_SUB, POS_PER_SUB)],
                        idx_v)

        def vec_pass(j, carry):
            r = idx_v[pl.ds(j * LANES, LANES)]
            rloc = r - c * HALF
            limit = HALF + c * TAIL_ROWS
            keep = (rloc >= 0) & (rloc < limit)
            west = lax.div(rloc, jnp.int32(WIN))
            rl = rloc - west * WIN
            w_eff = jnp.where(keep, west, DUMP_BUCKET)
            pos = j * LANES + lane
            entry = pos * 16384 + rl
            wv_v[pl.ds(j * LANES, LANES)] = w_eff
            ev_v[pl.ds(j * LANES, LANES)] = entry
            return carry

        lax.fori_loop(0, POS_PER_SUB // LANES, vec_pass, 0)

        def zero_b(b, carry):
            sm[b] = 0
            return carry
        lax.fori_loop(0, NUM_BUCKETS, zero_b, 0)

        def count_pos(j, carry):
            wvec = wv_v[pl.ds(j * LANES, LANES)]
            for jj in range(LANES):
                w = wvec[jj]
                sm[w] = sm[w] + 1
            return carry
        lax.fori_loop(0, POS_PER_SUB // LANES, count_pos, 0)

        ST = NUM_BUCKETS
        CU = NUM_BUCKETS + NUM_BUCKETS + 1
        sm[ST] = 0
        def pfx(b, carry):
            sm[ST + b + 1] = sm[ST + b] + sm[b]
            sm[CU + b] = sm[ST + b]
            return carry
        lax.fori_loop(0, NUM_BUCKETS, pfx, 0)

        LB = CU + NUM_BUCKETS
        def place_pos(j, carry):
            wvec = wv_v[pl.ds(j * LANES, LANES)]
            evec = ev_v[pl.ds(j * LANES, LANES)]
            for jj in range(LANES):
                w = wvec[jj]
                a = sm[CU + w]
                sm[LB + a] = evec[jj]
                sm[CU + w] = a + 1
            return carry
        lax.fori_loop(0, POS_PER_SUB // LANES, place_pos, 0)

        base_r = pl.multiple_of(c * HALF, 128)

        def do_fire(w, winbuf, sem):
            off = pl.multiple_of(base_r + w * WIN, 128)
            for u in range(4):
                d = s * 4 + u
                pltpu.async_copy(tabt.at[d, pl.ds(off, WIN)],
                                 winbuf.at[pl.ds(d * WIN, WIN)],
                                 sem)

        def wait_own(sem):
            for u in range(4):
                pltpu.make_async_copy(
                    tabt.at[0, pl.ds(0, WIN)],
                    win_a.at[pl.ds(0, WIN)], sem).wait()

        def extract(w, winbuf):
            n0 = sm[ST + w]
            n1 = sm[ST + w + 1]
            nch = (n1 - n0 + (CHUNK_E - 1)) // CHUNK_E

            def chunk(ck, carry):
                e0 = n0 + ck * CHUNK_E
                nthis = jnp.minimum(CHUNK_E, n1 - e0)

                def gfire(t, carry2):
                    pk = sm[LB + e0 + t]
                    p_loc = pk // 16384
                    rl = pk - p_loc * 16384
                    for k in range(4):
                        ib[pl.ds(t * EMBED + k * LANES, LANES)] = \
                            gbase[k] + rl
                    pltpu.async_copy(
                        winbuf.at[ib.at[pl.ds(t * EMBED, EMBED)]],
                        tmp.at[t], sem_g)
                    return carry2
                lax.fori_loop(0, nthis, gfire, 0)

                def gdrain(t, carry2):
                    pltpu.make_async_copy(
                        winbuf.at[ib.at[pl.ds(0, EMBED)]],
                        tmp.at[0], sem_g).wait()
                    return carry2
                lax.fori_loop(0, nthis, gdrain, 0)

                def wfire(t, carry2):
                    pk = sm[LB + e0 + t]
                    p_loc = pk // 16384
                    p_glob = s * POS_PER_SUB + p_loc
                    pltpu.async_copy(tmp.at[t], stage.at[p_glob], sem_w)
                    return carry2
                lax.fori_loop(0, nthis, wfire, 0)

                def wdrain(t, carry2):
                    pltpu.make_async_copy(
                        tmp.at[0], stage.at[0], sem_w).wait()
                    return carry2
                lax.fori_loop(0, nthis, wdrain, 0)
                return carry
            lax.fori_loop(0, nch, chunk, 0)


        def pair(q, carry):
            w0 = 2 * q
            wait_own(sem_a)
            plsc.subcore_barrier()
            extract(w0, win_a)
            plsc.subcore_barrier()
            do_fire(w0 + 2, win_a, sem_a)
            wait_own(sem_b)
            plsc.subcore_barrier()
            extract(w0 + 1, win_b)
            plsc.subcore_barrier()
            do_fire(w0 + 3, win_b, sem_b)
            return carry

        lax.fori_loop(0, NUM_WIN // 2 - 1, pair, 0)
        wait_own(sem_a)
        plsc.subcore_barrier()
        extract(NUM_WIN - 2, win_a)
        wait_own(sem_b)
        plsc.subcore_barrier()
        extract(NUM_WIN - 1, win_b)

        n0 = sm[ST + TAIL_BUCKET]
        n1 = sm[ST + TAIL_BUCKET + 1]

        def tail_ent(t, carry):
            pk = sm[LB + n0 + t]
            p_loc = pk // 16384
            rl = pk - p_loc * 16384
            p_glob = s * POS_PER_SUB + p_loc
            pltpu.async_copy(tailv.at[rl], stage.at[p_glob], sem_w)
            pltpu.make_async_copy(tailv.at[0], stage.at[0],
                                  sem_w).wait()
            return carry
        lax.fori_loop(0, n1 - n0, tail_ent, 0)

        plsc.subcore_barrier()


def _dot_body(stage_u, stage_i, out_hbm, rows_u, rows_i, out_v,
              sems_u, sems_i):
    wid = lax.axis_index("s") * NUM_CORES + lax.axis_index("c")
    base = wid * ROWS_PER_W2
    lane = lax.iota(jnp.int32, LANES)
    perms = [lane ^ sh for sh in (8, 4, 2, 1)]

    def dyn_perm(a, perm):
        return lax.gather(
            a, perm[:, None],
            lax.GatherDimensionNumbers(offset_dims=(),
                                       collapsed_slice_dims=(0,),
                                       start_index_map=(0,)),
            slice_sizes=(1,), mode=lax.GatherScatterMode.PROMISE_IN_BOUNDS)

    def start(cn):
        slot = cn % 2
        du = pltpu.async_copy(stage_u.at[pl.ds(base + cn * CHUNK2, CHUNK2)],
                              rows_u.at[slot], sems_u[slot])
        di = pltpu.async_copy(stage_i.at[pl.ds(base + cn * CHUNK2, CHUNK2)],
                              rows_i.at[slot], sems_i[slot])
        return du, di

    def compute(cn):
        slot = cn % 2
        ru = rows_u.at[slot]
        ri = rows_i.at[slot]

        def group(g, carry):
            res = jnp.zeros((LANES,), jnp.float32)
            for j in range(LANES):
                r = g * LANES + j
                acc = (ru[r, pl.ds(0, LANES)] * ri[r, pl.ds(0, LANES)]
                       + ru[r, pl.ds(LANES, LANES)]
                       * ri[r, pl.ds(LANES, LANES)])
                acc = acc + (ru[r, pl.ds(2 * LANES, LANES)]
                             * ri[r, pl.ds(2 * LANES, LANES)]
                             + ru[r, pl.ds(3 * LANES, LANES)]
                             * ri[r, pl.ds(3 * LANES, LANES)])
                for p in perms:
                    acc = acc + dyn_perm(acc, p)
                res = jnp.where(lane == j, acc, res)
            out_v[pl.ds(cn * CHUNK2 + g * LANES, LANES)] = res
            return carry

        lax.fori_loop(0, CHUNK2 // LANES, group, 0)

    descs = start(0)
    for cn in range(ROWS_PER_W2 // CHUNK2):
        nxt = start(cn + 1) if cn + 1 < ROWS_PER_W2 // CHUNK2 else None
        descs[0].wait()
        descs[1].wait()
        compute(cn)
        descs = nxt

    pltpu.sync_copy(out_v, out_hbm.at[pl.ds(base, ROWS_PER_W2)])


@jax.jit
def _scores(users, items, user_table, item_table):
    mesh = plsc.VectorSubcoreMesh(core_axis_name="c", subcore_axis_name="s",
                                  num_cores=NUM_CORES,
                                  num_subcores=NUM_SUBCORES)
    tabt_u = user_table.T
    tabt_i = item_table.T
    tail_u = lax.slice(user_table, (TAIL_BASE, 0), (NUM_ROWS, EMBED))
    tail_i = lax.slice(item_table, (TAIL_BASE, 0), (NUM_ROWS, EMBED))

    stage_u, stage_i = pl.kernel(
        _gather_body,
        out_type=(jax.ShapeDtypeStruct((BATCH, EMBED), jnp.float32),
                  jax.ShapeDtypeStruct((BATCH, EMBED), jnp.float32)),
        mesh=mesh,
        scratch_types=[
            pltpu.VMEM((POS_PER_SUB,), jnp.int32),
            pltpu.VMEM((POS_PER_SUB,), jnp.int32),
            pltpu.VMEM((POS_PER_SUB,), jnp.int32),
            pltpu.VMEM((TAIL_ROWS, EMBED), jnp.float32),
            pltpu.VMEM((CHUNK_E, EMBED), jnp.float32),
            pltpu.VMEM((CHUNK_E * EMBED,), jnp.int32),
            pltpu.VMEM_SHARED((EMBED * WIN,), jnp.float32),
            pltpu.VMEM_SHARED((EMBED * WIN,), jnp.float32),
            pltpu.SMEM((NUM_BUCKETS * 2 + 2 + POS_PER_SUB,), jnp.int32),
            pltpu.SemaphoreType.DMA,
            pltpu.SemaphoreType.DMA,
            pltpu.SemaphoreType.DMA,
            pltpu.SemaphoreType.DMA,
        ],
    )(users, items, tabt_u, tabt_i, tail_u, tail_i)

    return pl.kernel(
        _dot_body,
        out_type=jax.ShapeDtypeStruct((BATCH,), jnp.float32),
        mesh=mesh,
        scratch_types=[
            pltpu.VMEM((2, CHUNK2, EMBED), jnp.float32),
            pltpu.VMEM((2, CHUNK2, EMBED), jnp.float32),
            pltpu.VMEM((ROWS_PER_W2,), jnp.float32),
            [pltpu.SemaphoreType.DMA, pltpu.SemaphoreType.DMA],
            [pltpu.SemaphoreType.DMA, pltpu.SemaphoreType.DMA],
        ],
    )(stage_u, stage_i)


def kernel(users, items, user_table, item_table):
    return _scores(users, items, user_table, item_table)
